# baseline (speedup 1.0000x reference)
import functools

import jax
import jax.numpy as jnp
from jax import lax
from jax.experimental import pallas as pl
from jax.experimental.pallas import tpu as pltpu
from jax.experimental.pallas import tpu_sc as plsc

ALPHA = 0.2
BATCH = 16384
EMBED_DIM = 64
CHUNK = 128
NGROUP = CHUNK // 16


def _sc_body(e_hbm, rpad_hbm, s_hbm, r_hbm, o_hbm, out_hbm,
             s_i, o_i, r_i, srow, orow, rrow, outbuf, sems):
    info = plsc.get_sparse_core_info()
    nw = info.num_cores * info.num_subcores
    b_per_w = BATCH // nw
    nchunk = b_per_w // CHUNK

    wid = lax.axis_index("s") * info.num_cores + lax.axis_index("c")
    base = wid * b_per_w
    lanes = lax.iota(jnp.int32, 16)

    def issue(c, buf):
        off0 = base + c * CHUNK
        pltpu.sync_copy(s_hbm.at[pl.ds(off0, CHUNK)], s_i.at[buf])
        pltpu.sync_copy(o_hbm.at[pl.ds(off0, CHUNK)], o_i.at[buf])
        pltpu.sync_copy(r_hbm.at[pl.ds(off0, CHUNK)], r_i.at[buf])
        pltpu.async_copy(e_hbm.at[s_i.at[buf]], srow.at[buf], sems.at[buf])
        pltpu.async_copy(e_hbm.at[o_i.at[buf]], orow.at[buf], sems.at[buf])
        pltpu.async_copy(rpad_hbm.at[r_i.at[buf]], rrow.at[buf], sems.at[buf])

    def drain(buf):
        pltpu.make_async_copy(e_hbm.at[pl.ds(0, CHUNK)], srow.at[buf],
                              sems.at[buf]).wait()
        pltpu.make_async_copy(e_hbm.at[pl.ds(0, CHUNK)], orow.at[buf],
                              sems.at[buf]).wait()
        pltpu.make_async_copy(rpad_hbm.at[pl.ds(0, CHUNK)], rrow.at[buf],
                              sems.at[buf]).wait()

    def compute(c, buf):
        for bg in range(NGROUP):
            brow = bg * 16 + lanes

            def d_body(d, acc):
                dv = jnp.full((16,), 0, jnp.int32) + d
                sv = plsc.load_gather(srow.at[buf], [brow, dv])
                ov = plsc.load_gather(orow.at[buf], [brow, dv])
                rc = plsc.load_gather(rrow.at[buf], [brow, dv])
                ro = plsc.load_gather(rrow.at[buf],
                                      [brow, dv + EMBED_DIM])
                t = ov - sv - rc
                off = jnp.maximum(ro, 0.0)
                a = jnp.abs(t)
                return acc + (jnp.maximum(a - off, 0.0)
                              + ALPHA * jnp.minimum(a, off))

            acc = lax.fori_loop(0, EMBED_DIM, d_body,
                                jnp.zeros((16,), jnp.float32))
            outbuf[pl.ds(c * CHUNK + bg * 16, 16)] = -acc

    issue(0, 0)

    def chunk_body(c, carry):
        buf = c % 2

        @pl.when(c + 1 < nchunk)
        def _():
            issue(c + 1, 1 - buf)

        drain(buf)
        compute(c, buf)
        return carry

    lax.fori_loop(0, nchunk, chunk_body, 0)
    pltpu.sync_copy(outbuf, out_hbm.at[pl.ds(base, b_per_w)])


def kernel(E_center, R_center, R_offset, s, r, o):
    info = plsc.get_sparse_core_info()
    nw = info.num_cores * info.num_subcores
    b_per_w = BATCH // nw

    rpad = jnp.concatenate([R_center, R_offset], axis=1)

    run = functools.partial(
        pl.kernel,
        out_type=jax.ShapeDtypeStruct((BATCH,), jnp.float32),
        mesh=plsc.VectorSubcoreMesh(core_axis_name="c", subcore_axis_name="s"),
        compiler_params=pltpu.CompilerParams(
            needs_layout_passes=False, disable_bounds_checks=True,
            use_tc_tiling_on_sc=False),
        scratch_types=[
            pltpu.VMEM((2, CHUNK), jnp.int32),
            pltpu.VMEM((2, CHUNK), jnp.int32),
            pltpu.VMEM((2, CHUNK), jnp.int32),
            pltpu.VMEM((2, CHUNK, EMBED_DIM), jnp.float32),
            pltpu.VMEM((2, CHUNK, EMBED_DIM), jnp.float32),
            pltpu.VMEM((2, CHUNK, 2 * EMBED_DIM), jnp.float32),
            pltpu.VMEM((b_per_w,), jnp.float32),
            pltpu.SemaphoreType.DMA((2,)),
        ],
    )(_sc_body)

    return run(E_center, rpad,
               s.astype(jnp.int32), r.astype(jnp.int32), o.astype(jnp.int32))
